# Initial kernel scaffold; baseline (speedup 1.0000x reference)
import jax
import jax.numpy as jnp
from jax.experimental import pallas as pl


def kernel(new_k, new_v, k_buf, v_buf):
    raise NotImplementedError("write your pallas kernel here")


# speedup vs baseline: 3.1102x; 3.1102x over previous
import jax
import jax.numpy as jnp
from jax.experimental import pallas as pl


def _body(nk_ref, nv_ref, kb_ref, vb_ref, ok_ref, ov_ref):
    B = ok_ref.shape[0]
    W = ok_ref.shape[2]
    D = ok_ref.shape[3]
    S = nk_ref.shape[2]
    shifted_k = kb_ref[:, :, 1:, :]
    shifted_v = vb_ref[:, :, 1:, :]
    ok_ref[:, :, : W - 1, :] = jnp.broadcast_to(shifted_k, (B, 1, W - 1, D))
    ov_ref[:, :, : W - 1, :] = jnp.broadcast_to(shifted_v, (B, 1, W - 1, D))
    ok_ref[:, :, W - 1 :, :] = nk_ref[:, :, S - 1 :, :]
    ov_ref[:, :, W - 1 :, :] = nv_ref[:, :, S - 1 :, :]


def kernel(new_k, new_v, k_buf, v_buf):
    B, H, S, D = new_k.shape
    W = k_buf.shape[2]
    grid = (H,)
    out_shape = jax.ShapeDtypeStruct((B, H, W, D), new_k.dtype)
    updated_k, updated_v = pl.pallas_call(
        _body,
        grid=grid,
        in_specs=[
            pl.BlockSpec((B, 1, S, D), lambda h: (0, h, 0, 0)),
            pl.BlockSpec((B, 1, S, D), lambda h: (0, h, 0, 0)),
            pl.BlockSpec((1, 1, W, D), lambda h: (0, h, 0, 0)),
            pl.BlockSpec((1, 1, W, D), lambda h: (0, h, 0, 0)),
        ],
        out_specs=[
            pl.BlockSpec((B, 1, W, D), lambda h: (0, h, 0, 0)),
            pl.BlockSpec((B, 1, W, D), lambda h: (0, h, 0, 0)),
        ],
        out_shape=[out_shape, out_shape],
    )(new_k, new_v, k_buf, v_buf)
    return (updated_k, updated_v)
